# baseline (speedup 1.0000x reference)
import functools

import jax
import jax.numpy as jnp
import numpy as np
from jax import lax
from jax.experimental import pallas as pl
from jax.experimental.pallas import tpu as pltpu
from jax.experimental.pallas import tpu_sc as plsc

B, S, D, H = 1, 2048, 1024, 16
DH = D // H
HID = 3328
E = 8
EPS = 1e-06

EP = 128
T = 128
NT = 24
PADT = NT * T
NEG = -1e9

F32 = jnp.float32
I32 = jnp.int32


def _k1_body(x_ref, anw_ref, wq_ref, wk_ref, wv_ref, wqs_ref, wks_ref,
             q_ref, qs_ref, k_ref, ks_ref, v_ref):
    x = x_ref[...]
    h1 = x * lax.rsqrt(jnp.mean(x * x, axis=-1, keepdims=True) + EPS)
    h1 = h1 * anw_ref[...]
    q = jnp.dot(h1, wq_ref[...], preferred_element_type=F32)
    qs = jnp.dot(h1, wqs_ref[...], preferred_element_type=F32)
    k = jnp.dot(h1, wk_ref[...], preferred_element_type=F32)
    ks = jnp.dot(h1, wks_ref[...], preferred_element_type=F32)
    qi = lax.rsqrt(jnp.mean(q * q, axis=-1, keepdims=True) + EPS)
    ki = lax.rsqrt(jnp.mean(k * k, axis=-1, keepdims=True) + EPS)
    q_ref[...] = q * qi
    qs_ref[...] = qs * qi
    k_ref[...] = k * ki
    ks_ref[...] = ks * ki
    v_ref[...] = jnp.dot(h1, wv_ref[...], preferred_element_type=F32)


def _qkv(x2, anw, wq, wk, wv, wqs, wks):
    bt = 256
    grid = (S // bt,)
    spec_x = pl.BlockSpec((bt, D), lambda i: (i, 0))
    spec_w = pl.BlockSpec((D, D), lambda i: (0, 0))
    spec_n = pl.BlockSpec((1, D), lambda i: (0, 0))
    return pl.pallas_call(
        _k1_body,
        grid=grid,
        in_specs=[spec_x, spec_n, spec_w, spec_w, spec_w, spec_w, spec_w],
        out_specs=[spec_x] * 5,
        out_shape=[jax.ShapeDtypeStruct((S, D), F32)] * 5,
    )(x2, anw, wq, wk, wv, wqs, wks)


def _k2_body(q_ref, qs_ref, k_ref, ks_ref, v_ref, c_ref, s2_ref, o_ref):
    qt = pl.program_id(1)
    qbase = qt * 256
    cq = c_ref[pl.ds(qbase, 256), :]
    sq = s2_ref[pl.ds(qbase, 256), :]
    qr = q_ref[...] * cq + qs_ref[...] * sq
    ri = lax.broadcasted_iota(I32, (256, 256), 0)
    ci = lax.broadcasted_iota(I32, (256, 256), 1)
    dmask = jnp.where(ci > ri, F32(NEG), F32(0.0))
    outs = []
    for h in range(2):
        qh = qr[:, h * DH:(h + 1) * DH]

        def kv(j, h=h):
            rows = pl.ds(j * 256, 256)
            lanes = pl.ds(h * DH, DH)
            kh = (k_ref[rows, lanes] * c_ref[rows, lanes]
                  + ks_ref[rows, lanes] * s2_ref[rows, lanes])
            vh = v_ref[rows, lanes]
            return kh, vh

        def step(s, carry):
            m, l, acc, vh = carry
            mn = jnp.maximum(m, jnp.max(s, axis=-1, keepdims=True))
            p = jnp.exp(s - mn)
            corr = jnp.exp(m - mn)
            l = l * corr + jnp.sum(p, axis=-1, keepdims=True)
            acc = acc * corr + jnp.dot(p, vh, preferred_element_type=F32)
            return mn, l, acc

        def body(j, carry):
            kh, vh = kv(j)
            s = jax.lax.dot_general(
                qh, kh, (((1,), (1,)), ((), ())),
                preferred_element_type=F32) * 0.125
            return step(s, (*carry, vh))[:3]

        init = (jnp.full((256, 1), F32(NEG)), jnp.zeros((256, 1), F32),
                jnp.zeros((256, DH), F32))
        m, l, acc = lax.fori_loop(0, qt, body, init)
        kh, vh = kv(qt)
        s = jax.lax.dot_general(
            qh, kh, (((1,), (1,)), ((), ())),
            preferred_element_type=F32) * 0.125 + dmask
        m, l, acc = step(s, (m, l, acc, vh))
        outs.append(acc / l)
    o_ref[...] = jnp.concatenate(outs, axis=1)


def _attention(q, qs, k, ks, v, ch2, s2h2):
    grid = (H // 2, S // 256)
    spec_qt = pl.BlockSpec((256, 2 * DH), lambda hp, qt: (qt, hp))
    spec_full = pl.BlockSpec((S, 2 * DH), lambda hp, qt: (0, hp))
    spec_cs = pl.BlockSpec((S, 2 * DH), lambda hp, qt: (0, 0))
    return pl.pallas_call(
        _k2_body,
        grid=grid,
        in_specs=[spec_qt, spec_qt, spec_full, spec_full, spec_full,
                  spec_cs, spec_cs],
        out_specs=spec_qt,
        out_shape=jax.ShapeDtypeStruct((S, D), F32),
    )(q, qs, k, ks, v, ch2, s2h2)


def _k3_body(x_ref, ao_ref, wo_ref, fnw_ref, gw_ref,
             h_ref, t2_ref, wtop_ref, idx_ref):
    h = x_ref[...] + jnp.dot(ao_ref[...], wo_ref[...],
                             preferred_element_type=F32)
    h_ref[...] = h
    t2 = h * lax.rsqrt(jnp.mean(h * h, axis=-1, keepdims=True) + EPS)
    t2 = t2 * fnw_ref[...]
    t2_ref[...] = t2
    logits = jnp.dot(t2, gw_ref[...], preferred_element_type=F32)
    lane = lax.broadcasted_iota(I32, logits.shape, 1)
    logits = jnp.where(lane < E, logits, F32(NEG))
    m = jnp.max(logits, axis=-1, keepdims=True)
    p = jnp.exp(logits - m)
    probs = p / jnp.sum(p, axis=-1, keepdims=True)
    wtop = jnp.max(probs, axis=-1, keepdims=True)
    wtop_ref[...] = wtop
    idx_ref[...] = jnp.min(jnp.where(probs >= wtop, lane, I32(EP - 1)),
                           axis=-1, keepdims=True)


def _post_attn(x2, ao2, wo, fnw, gwp):
    bt = 256
    grid = (S // bt,)
    spec_x = pl.BlockSpec((bt, D), lambda i: (i, 0))
    spec_w = pl.BlockSpec((D, D), lambda i: (0, 0))
    spec_n = pl.BlockSpec((1, D), lambda i: (0, 0))
    spec_g = pl.BlockSpec((D, EP), lambda i: (0, 0))
    spec_c1 = pl.BlockSpec((bt, 1), lambda i: (i, 0))
    return pl.pallas_call(
        _k3_body,
        grid=grid,
        in_specs=[spec_x, spec_x, spec_w, spec_n, spec_g],
        out_specs=[spec_x, spec_x, spec_c1, spec_c1],
        out_shape=[jax.ShapeDtypeStruct((S, D), F32),
                   jax.ShapeDtypeStruct((S, D), F32),
                   jax.ShapeDtypeStruct((S, 1), F32),
                   jax.ShapeDtypeStruct((S, 1), I32)],
    )(x2, ao2, wo, fnw, gwp)


def _kr_body(idx_ref, dest_ref, te_ref, live_ref):
    idx = idx_ref[...]
    lane = lax.broadcasted_iota(I32, (S, EP), 1)
    oh = (idx == lane).astype(F32)
    r = lax.broadcasted_iota(I32, (S, S), 0)
    c = lax.broadcasted_iota(I32, (S, S), 1)
    tril = (r >= c).astype(F32)
    cum = jnp.dot(tril, oh, preferred_element_type=F32)
    rank = jnp.sum(oh * cum, axis=-1, keepdims=True)
    counts = jnp.sum(oh, axis=0, keepdims=True)
    ci = counts.astype(I32)
    pci = ((ci + (T - 1)) // T) * T
    pcf = pci.astype(F32)
    a = lax.broadcasted_iota(I32, (EP, EP), 0)
    b = lax.broadcasted_iota(I32, (EP, EP), 1)
    lmat = (a < b).astype(F32)
    off = jnp.dot(pcf, lmat, preferred_element_type=F32)
    dest = jnp.sum(oh * off, axis=-1, keepdims=True) + rank - 1.0
    dest_ref[...] = dest.astype(I32)
    ends = off + pcf
    rends = off + counts
    tlane = lax.broadcasted_iota(I32, (NT, EP), 1)
    st = (lax.broadcasted_iota(I32, (NT, 1), 0) * T).astype(F32)
    te = jnp.sum(jnp.where((ends <= st) & (tlane < E), 1, 0),
                 axis=-1, keepdims=True)
    te = jnp.minimum(te, E - 1)
    te_ref[...] = te
    oht = (te == tlane).astype(F32)
    myend = jnp.sum(oht * rends, axis=-1, keepdims=True)
    live_ref[...] = (st < myend).astype(I32)


def _routing(idxc):
    return pl.pallas_call(
        _kr_body,
        grid=(1,),
        in_specs=[pl.BlockSpec((S, 1), lambda i: (0, 0))],
        out_specs=[pl.BlockSpec((S, 1), lambda i: (0, 0)),
                   pl.BlockSpec((NT, 1), lambda i: (0, 0)),
                   pl.BlockSpec((NT, 1), lambda i: (0, 0))],
        out_shape=[jax.ShapeDtypeStruct((S, 1), I32),
                   jax.ShapeDtypeStruct((NT, 1), I32),
                   jax.ShapeDtypeStruct((NT, 1), I32)],
    )(idxc)


_NC, _NS = 2, 16
_NW = _NC * _NS
_CH = S // _NW


@functools.cache
def _sc_kernels():
    mesh = plsc.VectorSubcoreMesh(core_axis_name="c", subcore_axis_name="s")

    @functools.partial(
        pl.kernel, mesh=mesh,
        out_type=jax.ShapeDtypeStruct((PADT, D), F32),
        scratch_types=[pltpu.VMEM((_CH,), I32),
                       pltpu.VMEM((_CH, D), F32),
                       pltpu.SemaphoreType.DMA],
    )
    def sc_dispatch(t2_hbm, dest_hbm, xs_hbm, idx_v, rows_v, sem):
        wid = lax.axis_index("s") * _NC + lax.axis_index("c")
        base = wid * _CH
        pltpu.sync_copy(dest_hbm.at[pl.ds(base, _CH)], idx_v)
        pltpu.sync_copy(t2_hbm.at[pl.ds(base, _CH)], rows_v)
        pltpu.async_copy(rows_v, xs_hbm.at[idx_v], sem).wait()

    @functools.partial(
        pl.kernel, mesh=mesh,
        out_type=jax.ShapeDtypeStruct((S, D), F32),
        scratch_types=[pltpu.VMEM((_CH,), I32),
                       pltpu.VMEM((_CH, D), F32),
                       pltpu.SemaphoreType.DMA],
    )
    def sc_combine(ys_hbm, dest_hbm, moe_hbm, idx_v, rows_v, sem):
        wid = lax.axis_index("s") * _NC + lax.axis_index("c")
        base = wid * _CH
        pltpu.sync_copy(dest_hbm.at[pl.ds(base, _CH)], idx_v)
        pltpu.async_copy(ys_hbm.at[idx_v], rows_v, sem).wait()
        pltpu.sync_copy(rows_v, moe_hbm.at[pl.ds(base, _CH)])

    return sc_dispatch, sc_combine


def _k5_body(te_ref, live_ref, xs_ref, w1_ref, w3_ref, w2_ref, ys_ref):
    t = pl.program_id(0)

    @pl.when(live_ref[t] == 1)
    def _():
        x = xs_ref[...].astype(jnp.bfloat16)
        a1 = jnp.dot(x, w1_ref[0], preferred_element_type=F32)
        a3 = jnp.dot(x, w3_ref[0], preferred_element_type=F32)
        g = (a1 * jax.nn.sigmoid(a1) * a3).astype(jnp.bfloat16)
        ys_ref[...] = jnp.dot(g, w2_ref[0], preferred_element_type=F32)


def _gmm(te, live, xs, w1, w3, w2):
    grid_spec = pltpu.PrefetchScalarGridSpec(
        num_scalar_prefetch=2,
        grid=(NT,),
        in_specs=[
            pl.BlockSpec((T, D), lambda t, te, lv: (t, 0)),
            pl.BlockSpec((1, D, HID), lambda t, te, lv: (te[t], 0, 0)),
            pl.BlockSpec((1, D, HID), lambda t, te, lv: (te[t], 0, 0)),
            pl.BlockSpec((1, HID, D), lambda t, te, lv: (te[t], 0, 0)),
        ],
        out_specs=pl.BlockSpec((T, D), lambda t, te, lv: (t, 0)),
    )
    return pl.pallas_call(
        _k5_body,
        grid_spec=grid_spec,
        out_shape=jax.ShapeDtypeStruct((PADT, D), F32),
    )(te, live, xs, w1, w3, w2)


def _k7_body(t2_ref, w1_ref, w3_ref, w2_ref, h_ref, moe_ref, wtop_ref, o_ref):
    x = t2_ref[...].astype(jnp.bfloat16)
    a1 = jnp.dot(x, w1_ref[...], preferred_element_type=F32)
    a3 = jnp.dot(x, w3_ref[...], preferred_element_type=F32)
    g = (a1 * jax.nn.sigmoid(a1) * a3).astype(jnp.bfloat16)
    part = jnp.dot(g, w2_ref[...], preferred_element_type=F32)
    o_ref[...] = h_ref[...] + moe_ref[...] * wtop_ref[...] + part


def _shared_final(t2, sw1, sw3, sw2, h2, moe, wtop):
    bt = 256
    grid = (S // bt,)
    spec_x = pl.BlockSpec((bt, D), lambda i: (i, 0))
    spec_c1 = pl.BlockSpec((bt, 1), lambda i: (i, 0))
    return pl.pallas_call(
        _k7_body,
        grid=grid,
        in_specs=[spec_x,
                  pl.BlockSpec((D, HID), lambda i: (0, 0)),
                  pl.BlockSpec((D, HID), lambda i: (0, 0)),
                  pl.BlockSpec((HID, D), lambda i: (0, 0)),
                  spec_x, spec_x, spec_c1],
        out_specs=spec_x,
        out_shape=jax.ShapeDtypeStruct((S, D), F32),
    )(t2, sw1, sw3, sw2, h2, moe, wtop)


def kernel(x, start_pos, freqs_cos, freqs_sin, mask, wq, wk, wv, wo,
           attn_norm_w, ffn_norm_w, gate_w, exp_w1, exp_w2, exp_w3,
           sh_w1, sh_w2, sh_w3):
    x2 = x.reshape(S, D)
    anw = attn_norm_w.reshape(1, D)
    fnw = ffn_norm_w.reshape(1, D)
    gwp = jnp.pad(gate_w, ((0, 0), (0, EP - E)))

    perm = np.arange(D) ^ 1
    wqs = wq[:, perm]
    wks = wk[:, perm]

    q, qs, k, ks, v = _qkv(x2, anw, wq, wk, wv, wqs, wks)

    ch = jnp.repeat(freqs_cos, 2, axis=1)
    s2h = jnp.stack([-freqs_sin, freqs_sin], axis=-1).reshape(S, DH)
    ch2 = jnp.concatenate([ch, ch], axis=1)
    s2h2 = jnp.concatenate([s2h, s2h], axis=1)

    ao2 = _attention(q, qs, k, ks, v, ch2, s2h2)

    h2, t2, wtop, idxc = _post_attn(x2, ao2, wo, fnw, gwp)
    destc, tec, livec = _routing(idxc)
    dest = destc.reshape(S)
    te = tec.reshape(NT)
    live = livec.reshape(NT)

    sc_dispatch, sc_combine = _sc_kernels()
    xs = sc_dispatch(t2, dest)
    bf = jnp.bfloat16
    ys = _gmm(te, live, xs, exp_w1.astype(bf), exp_w3.astype(bf),
              exp_w2.astype(bf))
    moe = sc_combine(ys, dest)

    out = _shared_final(t2, sh_w1.astype(bf), sh_w3.astype(bf),
                        sh_w2.astype(bf), h2, moe, wtop)
    return out.reshape(B, S, D)


# speedup vs baseline: 1.3235x; 1.3235x over previous
import functools

import jax
import jax.numpy as jnp
import numpy as np
from jax import lax
from jax.experimental import pallas as pl
from jax.experimental.pallas import tpu as pltpu
from jax.experimental.pallas import tpu_sc as plsc

B, S, D, H = 1, 2048, 1024, 16
DH = D // H
HID = 3328
E = 8
EPS = 1e-06

EP = 128
T = 128
NT = 24
PADT = NT * T
NEG = -1e9

F32 = jnp.float32
I32 = jnp.int32


def _k1_body(x_ref, anw_ref, wq_ref, wk_ref, wv_ref, wqs_ref, wks_ref,
             q_ref, qs_ref, k_ref, ks_ref, v_ref):
    x = x_ref[...]
    h1 = x * lax.rsqrt(jnp.mean(x * x, axis=-1, keepdims=True) + EPS)
    h1 = h1 * anw_ref[...]
    q = jnp.dot(h1, wq_ref[...], preferred_element_type=F32)
    qs = jnp.dot(h1, wqs_ref[...], preferred_element_type=F32)
    k = jnp.dot(h1, wk_ref[...], preferred_element_type=F32)
    ks = jnp.dot(h1, wks_ref[...], preferred_element_type=F32)
    qi = lax.rsqrt(jnp.mean(q * q, axis=-1, keepdims=True) + EPS)
    ki = lax.rsqrt(jnp.mean(k * k, axis=-1, keepdims=True) + EPS)
    q_ref[...] = q * qi
    qs_ref[...] = qs * qi
    k_ref[...] = k * ki
    ks_ref[...] = ks * ki
    v_ref[...] = jnp.dot(h1, wv_ref[...], preferred_element_type=F32)


def _qkv(x2, anw, wq, wk, wv, wqs, wks):
    bt = 256
    grid = (S // bt,)
    spec_x = pl.BlockSpec((bt, D), lambda i: (i, 0))
    spec_w = pl.BlockSpec((D, D), lambda i: (0, 0))
    spec_n = pl.BlockSpec((1, D), lambda i: (0, 0))
    return pl.pallas_call(
        _k1_body,
        grid=grid,
        in_specs=[spec_x, spec_n, spec_w, spec_w, spec_w, spec_w, spec_w],
        out_specs=[spec_x] * 5,
        out_shape=[jax.ShapeDtypeStruct((S, D), F32)] * 5,
    )(x2, anw, wq, wk, wv, wqs, wks)


def _k2_body(q_ref, qs_ref, k_ref, ks_ref, v_ref, c_ref, s2_ref, o_ref):
    qt = pl.program_id(1)
    qbase = qt * 256
    cq = c_ref[pl.ds(qbase, 256), :]
    sq = s2_ref[pl.ds(qbase, 256), :]
    qr = q_ref[...] * cq + qs_ref[...] * sq
    kr = k_ref[...] * c_ref[...] + ks_ref[...] * s2_ref[...]
    row = qbase + lax.broadcasted_iota(I32, (256, S), 0)
    col = lax.broadcasted_iota(I32, (256, S), 1)
    cmask = jnp.where(col > row, F32(NEG), F32(0.0))
    outs = []
    for h in range(2):
        qh = qr[:, h * DH:(h + 1) * DH]
        kh = kr[:, h * DH:(h + 1) * DH]
        vh = v_ref[:, h * DH:(h + 1) * DH]
        s = jax.lax.dot_general(
            qh, kh, (((1,), (1,)), ((), ())),
            preferred_element_type=F32) * 0.125 + cmask
        m = jnp.max(s, axis=-1, keepdims=True)
        p = jnp.exp(s - m)
        attn = p / jnp.sum(p, axis=-1, keepdims=True)
        outs.append(jnp.dot(attn, vh, preferred_element_type=F32))
    o_ref[...] = jnp.concatenate(outs, axis=1)


def _attention(q, qs, k, ks, v, ch2, s2h2):
    grid = (H // 2, S // 256)
    spec_qt = pl.BlockSpec((256, 2 * DH), lambda hp, qt: (qt, hp))
    spec_full = pl.BlockSpec((S, 2 * DH), lambda hp, qt: (0, hp))
    spec_cs = pl.BlockSpec((S, 2 * DH), lambda hp, qt: (0, 0))
    return pl.pallas_call(
        _k2_body,
        grid=grid,
        in_specs=[spec_qt, spec_qt, spec_full, spec_full, spec_full,
                  spec_cs, spec_cs],
        out_specs=spec_qt,
        out_shape=jax.ShapeDtypeStruct((S, D), F32),
    )(q, qs, k, ks, v, ch2, s2h2)


def _k3_body(x_ref, ao_ref, wo_ref, fnw_ref, gw_ref,
             h_ref, t2_ref, wtop_ref, idx_ref):
    h = x_ref[...] + jnp.dot(ao_ref[...], wo_ref[...],
                             preferred_element_type=F32)
    h_ref[...] = h
    t2 = h * lax.rsqrt(jnp.mean(h * h, axis=-1, keepdims=True) + EPS)
    t2 = t2 * fnw_ref[...]
    t2_ref[...] = t2
    logits = jnp.dot(t2, gw_ref[...], preferred_element_type=F32)
    lane = lax.broadcasted_iota(I32, logits.shape, 1)
    logits = jnp.where(lane < E, logits, F32(NEG))
    m = jnp.max(logits, axis=-1, keepdims=True)
    p = jnp.exp(logits - m)
    probs = p / jnp.sum(p, axis=-1, keepdims=True)
    wtop = jnp.max(probs, axis=-1, keepdims=True)
    wtop_ref[...] = wtop
    idx_ref[...] = jnp.min(jnp.where(probs >= wtop, lane, I32(EP - 1)),
                           axis=-1, keepdims=True)


def _post_attn(x2, ao2, wo, fnw, gwp):
    bt = 256
    grid = (S // bt,)
    spec_x = pl.BlockSpec((bt, D), lambda i: (i, 0))
    spec_w = pl.BlockSpec((D, D), lambda i: (0, 0))
    spec_n = pl.BlockSpec((1, D), lambda i: (0, 0))
    spec_g = pl.BlockSpec((D, EP), lambda i: (0, 0))
    spec_c1 = pl.BlockSpec((bt, 1), lambda i: (i, 0))
    return pl.pallas_call(
        _k3_body,
        grid=grid,
        in_specs=[spec_x, spec_x, spec_w, spec_n, spec_g],
        out_specs=[spec_x, spec_x, spec_c1, spec_c1],
        out_shape=[jax.ShapeDtypeStruct((S, D), F32),
                   jax.ShapeDtypeStruct((S, D), F32),
                   jax.ShapeDtypeStruct((S, 1), F32),
                   jax.ShapeDtypeStruct((S, 1), I32)],
    )(x2, ao2, wo, fnw, gwp)


def _kr_body(idx_ref, dest_ref, te_ref, live_ref):
    idx = idx_ref[...]
    lane = lax.broadcasted_iota(I32, (S, EP), 1)
    oh = (idx == lane).astype(F32)
    r = lax.broadcasted_iota(I32, (S, S), 0)
    c = lax.broadcasted_iota(I32, (S, S), 1)
    tril = (r >= c).astype(F32)
    cum = jnp.dot(tril, oh, preferred_element_type=F32)
    rank = jnp.sum(oh * cum, axis=-1, keepdims=True)
    counts = jnp.sum(oh, axis=0, keepdims=True)
    ci = counts.astype(I32)
    pci = ((ci + (T - 1)) // T) * T
    pcf = pci.astype(F32)
    a = lax.broadcasted_iota(I32, (EP, EP), 0)
    b = lax.broadcasted_iota(I32, (EP, EP), 1)
    lmat = (a < b).astype(F32)
    off = jnp.dot(pcf, lmat, preferred_element_type=F32)
    dest = jnp.sum(oh * off, axis=-1, keepdims=True) + rank - 1.0
    dest_ref[...] = dest.astype(I32)
    ends = off + pcf
    rends = off + counts
    tlane = lax.broadcasted_iota(I32, (NT, EP), 1)
    st = (lax.broadcasted_iota(I32, (NT, 1), 0) * T).astype(F32)
    te = jnp.sum(jnp.where((ends <= st) & (tlane < E), 1, 0),
                 axis=-1, keepdims=True)
    te = jnp.minimum(te, E - 1)
    te_ref[...] = te
    oht = (te == tlane).astype(F32)
    myend = jnp.sum(oht * rends, axis=-1, keepdims=True)
    live_ref[...] = (st < myend).astype(I32)


def _routing(idxc):
    return pl.pallas_call(
        _kr_body,
        grid=(1,),
        in_specs=[pl.BlockSpec((S, 1), lambda i: (0, 0))],
        out_specs=[pl.BlockSpec((S, 1), lambda i: (0, 0)),
                   pl.BlockSpec((NT, 1), lambda i: (0, 0)),
                   pl.BlockSpec((NT, 1), lambda i: (0, 0))],
        out_shape=[jax.ShapeDtypeStruct((S, 1), I32),
                   jax.ShapeDtypeStruct((NT, 1), I32),
                   jax.ShapeDtypeStruct((NT, 1), I32)],
    )(idxc)


_NC, _NS = 2, 16
_NW = _NC * _NS
_CH = S // _NW


@functools.cache
def _sc_kernels():
    mesh = plsc.VectorSubcoreMesh(core_axis_name="c", subcore_axis_name="s")

    @functools.partial(
        pl.kernel, mesh=mesh,
        out_type=jax.ShapeDtypeStruct((PADT, D), F32),
        scratch_types=[pltpu.VMEM((_CH,), I32),
                       pltpu.VMEM((_CH, D), F32),
                       pltpu.SemaphoreType.DMA],
    )
    def sc_dispatch(t2_hbm, dest_hbm, xs_hbm, idx_v, rows_v, sem):
        wid = lax.axis_index("s") * _NC + lax.axis_index("c")
        base = wid * _CH
        pltpu.sync_copy(dest_hbm.at[pl.ds(base, _CH)], idx_v)
        pltpu.sync_copy(t2_hbm.at[pl.ds(base, _CH)], rows_v)
        pltpu.async_copy(rows_v, xs_hbm.at[idx_v], sem).wait()

    @functools.partial(
        pl.kernel, mesh=mesh,
        out_type=[jax.ShapeDtypeStruct((S, D), F32),
                  jax.ShapeDtypeStruct((S, D), F32)],
        scratch_types=[pltpu.VMEM((_CH,), I32),
                       pltpu.VMEM((_CH, D), F32),
                       pltpu.SemaphoreType.DMA],
    )
    def sc_combine(ys_hbm, dest_hbm, moe0_hbm, moe1_hbm, idx_v, rows_v, sem):
        wid = lax.axis_index("s") * _NC + lax.axis_index("c")
        base = wid * _CH
        pltpu.sync_copy(dest_hbm.at[pl.ds(base, _CH)], idx_v)
        pltpu.async_copy(ys_hbm.at[idx_v], rows_v, sem).wait()
        pltpu.sync_copy(rows_v, moe0_hbm.at[pl.ds(base, _CH)])
        for i in range(_CH // 16):
            sl = pl.ds(i * 16, 16)
            idx_v[sl] = idx_v[sl] + PADT
        pltpu.async_copy(ys_hbm.at[idx_v], rows_v, sem).wait()
        pltpu.sync_copy(rows_v, moe1_hbm.at[pl.ds(base, _CH)])

    return sc_dispatch, sc_combine


KH5 = 2
HC5 = HID // KH5


def _k5_body(te_ref, live_ref, xs_ref, w1_ref, w3_ref, w2_ref, ys_ref):
    t = pl.program_id(1)

    @pl.when(live_ref[t] == 1)
    def _():
        x = xs_ref[...]
        a1 = jnp.dot(x, w1_ref[0], preferred_element_type=F32)
        a3 = jnp.dot(x, w3_ref[0], preferred_element_type=F32)
        g = a1 * jax.nn.sigmoid(a1) * a3
        ys_ref[0] = jnp.dot(g, w2_ref[0], preferred_element_type=F32)


def _gmm(te, live, xs, w1, w3, w2):
    grid_spec = pltpu.PrefetchScalarGridSpec(
        num_scalar_prefetch=2,
        grid=(KH5, NT),
        in_specs=[
            pl.BlockSpec((T, D), lambda k, t, te, lv: (t, 0)),
            pl.BlockSpec((1, D, HC5), lambda k, t, te, lv: (te[t], 0, k)),
            pl.BlockSpec((1, D, HC5), lambda k, t, te, lv: (te[t], 0, k)),
            pl.BlockSpec((1, HC5, D), lambda k, t, te, lv: (te[t], k, 0)),
        ],
        out_specs=pl.BlockSpec((1, T, D), lambda k, t, te, lv: (k, t, 0)),
    )
    return pl.pallas_call(
        _k5_body,
        grid_spec=grid_spec,
        out_shape=jax.ShapeDtypeStruct((KH5, PADT, D), F32),
    )(te, live, xs, w1, w3, w2)


def _k7_body(t2_ref, w1_ref, w3_ref, w2_ref, h_ref, moe0_ref, moe1_ref,
             wtop_ref, o_ref):
    x = t2_ref[...].astype(jnp.bfloat16)
    a1 = jnp.dot(x, w1_ref[...], preferred_element_type=F32)
    a3 = jnp.dot(x, w3_ref[...], preferred_element_type=F32)
    g = (a1 * jax.nn.sigmoid(a1) * a3).astype(jnp.bfloat16)
    part = jnp.dot(g, w2_ref[...], preferred_element_type=F32)
    moe = moe0_ref[...] + moe1_ref[...]
    o_ref[...] = h_ref[...] + moe * wtop_ref[...] + part


def _shared_final(t2, sw1, sw3, sw2, h2, moe0, moe1, wtop):
    bt = 256
    grid = (S // bt,)
    spec_x = pl.BlockSpec((bt, D), lambda i: (i, 0))
    spec_c1 = pl.BlockSpec((bt, 1), lambda i: (i, 0))
    return pl.pallas_call(
        _k7_body,
        grid=grid,
        in_specs=[spec_x,
                  pl.BlockSpec((D, HID), lambda i: (0, 0)),
                  pl.BlockSpec((D, HID), lambda i: (0, 0)),
                  pl.BlockSpec((HID, D), lambda i: (0, 0)),
                  spec_x, spec_x, spec_x, spec_c1],
        out_specs=spec_x,
        out_shape=jax.ShapeDtypeStruct((S, D), F32),
    )(t2, sw1, sw3, sw2, h2, moe0, moe1, wtop)


def kernel(x, start_pos, freqs_cos, freqs_sin, mask, wq, wk, wv, wo,
           attn_norm_w, ffn_norm_w, gate_w, exp_w1, exp_w2, exp_w3,
           sh_w1, sh_w2, sh_w3):
    x2 = x.reshape(S, D)
    anw = attn_norm_w.reshape(1, D)
    fnw = ffn_norm_w.reshape(1, D)
    gwp = jnp.pad(gate_w, ((0, 0), (0, EP - E)))

    perm = np.arange(D) ^ 1
    wqs = wq[:, perm]
    wks = wk[:, perm]

    q, qs, k, ks, v = _qkv(x2, anw, wq, wk, wv, wqs, wks)

    ch = jnp.repeat(freqs_cos, 2, axis=1)
    s2h = jnp.stack([-freqs_sin, freqs_sin], axis=-1).reshape(S, DH)
    ch2 = jnp.concatenate([ch, ch], axis=1)
    s2h2 = jnp.concatenate([s2h, s2h], axis=1)

    ao2 = _attention(q, qs, k, ks, v, ch2, s2h2)

    h2, t2, wtop, idxc = _post_attn(x2, ao2, wo, fnw, gwp)
    destc, tec, livec = _routing(idxc)
    dest = destc.reshape(S)
    te = tec.reshape(NT)
    live = livec.reshape(NT)

    sc_dispatch, sc_combine = _sc_kernels()
    xs = sc_dispatch(t2, dest)
    ys = _gmm(te, live, xs, exp_w1, exp_w3, exp_w2)
    moe0, moe1 = sc_combine(ys.reshape(KH5 * PADT, D), dest)

    bf = jnp.bfloat16
    out = _shared_final(t2, sh_w1.astype(bf), sh_w3.astype(bf),
                        sh_w2.astype(bf), h2, moe0, moe1, wtop)
    return out.reshape(B, S, D)
